# Initial kernel scaffold; baseline (speedup 1.0000x reference)
import jax
import jax.numpy as jnp
from jax.experimental import pallas as pl


def kernel(edge_index, Wq1, bq1, Wq2, bq2, Wc1, bc1, Wc2, bc2, Wu1, bu1, Wu2, bu2, Wu3, bu3, Wo1, bo1, Wo2, bo2):
    raise NotImplementedError("write your pallas kernel here")


# speedup vs baseline: 1.9790x; 1.9790x over previous
import functools

import jax
import jax.numpy as jnp
from jax.experimental import pallas as pl
from jax.experimental.pallas import tpu as pltpu

N_VARS = 25000
N_CLAUSES = 50000
N_EDGES = 800000
FEATURE_MAPS = 64
QUERY_MAPS = 64
TEST_ROUNDS = 8
EPS = 1e-6


def _adj_matmul(lit_idx, cls_idx, X):
    return jax.ops.segment_sum(jnp.take(X, cls_idx, axis=0), lit_idx, num_segments=2 * N_VARS)


def _cl_adj_matmul(lit_idx, cls_idx, Y):
    return jax.ops.segment_sum(jnp.take(Y, lit_idx, axis=0), cls_idx, num_segments=N_CLAUSES)


def _pair_norm(x):
    x = x - jnp.mean(x, axis=0, keepdims=True)
    var = jnp.mean(jnp.sum(jnp.square(x), axis=-1))
    return x * jax.lax.rsqrt(var + EPS)


def _mlp2(x, W1, b1, W2, b2):
    return jax.nn.relu(x @ W1 + b1) @ W2 + b2


def _mlp3(x, W1, b1, W2, b2, W3, b3):
    h = jax.nn.relu(x @ W1 + b1)
    h = jax.nn.relu(h @ W2 + b2)
    return h @ W3 + b3


def _mixed_loss_kernel(s8_ref, out_ref):
    cv = jnp.exp(-s8_ref[...])
    mixed = cv * (-jnp.log(1.0 - cv + EPS))
    out_ref[...] = jnp.sum(mixed, axis=1, keepdims=True) / float(TEST_ROUNDS)


def _mixed_loss(S8):
    blk = 5000
    return pl.pallas_call(
        _mixed_loss_kernel,
        grid=(N_CLAUSES // blk,),
        in_specs=[pl.BlockSpec((blk, 8), lambda i: (i, 0))],
        out_specs=pl.BlockSpec((blk, 1), lambda i: (i, 0)),
        out_shape=jax.ShapeDtypeStruct((N_CLAUSES, 1), jnp.float32),
    )(S8)


def kernel(edge_index, Wq1, bq1, Wq2, bq2, Wc1, bc1, Wc2, bc2, Wu1, bu1, Wu2, bu2, Wu3, bu3, Wo1, bo1, Wo2, bo2):
    lit_idx = edge_index[0]
    cls_idx = edge_index[1]
    variables = jnp.ones((N_VARS, FEATURE_MAPS), jnp.float32)
    clause_state = jnp.ones((N_CLAUSES, FEATURE_MAPS), jnp.float32)
    lit_degree = jax.ops.segment_sum(jnp.ones((N_EDGES,), jnp.float32), lit_idx, num_segments=2 * N_VARS)[:, None]
    degree_weight = jax.lax.rsqrt(jnp.maximum(lit_degree, 1.0))
    var_degree_weight = 4.0 * jax.lax.rsqrt(jnp.maximum(lit_degree[:N_VARS] + lit_degree[N_VARS:], 1.0))
    nkey = jax.random.key(42)
    last_logits = jnp.zeros((N_VARS, 1), jnp.float32)
    l8_cols = []
    for step in range(TEST_ROUNDS):
        noise = jax.random.normal(jax.random.fold_in(nkey, step), (N_VARS, 4), dtype=jnp.float32)
        v1 = jnp.concatenate([variables, noise], axis=-1)
        query = _mlp2(v1, Wq1, bq1, Wq2, bq2)
        lits = jax.nn.softplus(jnp.concatenate([query, -query], axis=0))
        S = _cl_adj_matmul(lit_idx, cls_idx, lits)
        cl = jnp.exp(-S)
        clauses_loss = cl * 4.0
        clause_unit = jnp.concatenate([clause_state, clauses_loss], axis=-1)
        clause_data = _mlp2(clause_unit, Wc1, bc1, Wc2, bc2)
        variables_loss_all = clause_data[:, :QUERY_MAPS]
        new_clause_value = _pair_norm(clause_data[:, QUERY_MAPS:]) * 0.25
        clause_state = new_clause_value + 0.1 * clause_state
        M = jnp.concatenate([cl, variables_loss_all], axis=-1)
        R = _adj_matmul(lit_idx, cls_idx, M)
        g = -R[:, :FEATURE_MAPS]
        variables_grad = (g[:N_VARS] * jax.nn.sigmoid(query)
                          - g[N_VARS:] * jax.nn.sigmoid(-query)) * var_degree_weight
        variables_loss = R[:, FEATURE_MAPS:] * degree_weight
        unit = jnp.concatenate(
            [variables_grad, variables, variables_loss[:N_VARS], variables_loss[N_VARS:]], axis=-1)
        new_variables = _pair_norm(_mlp3(unit, Wu1, bu1, Wu2, bu2, Wu3, bu3)) * 0.25
        variables = new_variables + 0.1 * variables
        logits = _mlp2(variables, Wo1, bo1, Wo2, bo2)
        l8_cols.append(jax.nn.softplus(jnp.concatenate([logits, -logits], axis=0))[:, 0])
        last_logits = logits
    L8 = jnp.stack(l8_cols, axis=1)
    S8 = _cl_adj_matmul(lit_idx, cls_idx, L8)
    unsupervised_loss = _mixed_loss(S8)
    step = jnp.asarray(TEST_ROUNDS - 1, dtype=jnp.int32)
    return (last_logits, unsupervised_loss, step)


# speedup vs baseline: 4.9154x; 2.4838x over previous
import functools

import jax
import jax.numpy as jnp
from jax import lax
from jax.experimental import pallas as pl
from jax.experimental.pallas import tpu as pltpu
from jax.experimental.pallas import tpu_sc as plsc

N_VARS = 25000
N_CLAUSES = 50000
N_EDGES = 800000
FEATURE_MAPS = 64
QUERY_MAPS = 64
TEST_ROUNDS = 8
EPS = 1e-6

NSEG = 50000
E_PAD = 802816
CHUNK = 128
NJ16 = E_PAD // (16 * CHUNK)
NJ32 = E_PAD // (32 * CHUNK)
ACC_ROWS = 50176
ZROWS = ACC_ROWS // 16
OUT_ROWS = 50048
OROWS = OUT_ROWS // 16


def _segsum_slab_body(n_passes, d):
    def body(src_idx, dst_idx, table, zeros, out, acc, sidx, didx, rows, sem):
        c = lax.axis_index("c")
        s = lax.axis_index("s")
        for p in range(n_passes):
            pltpu.sync_copy(zeros, acc.at[pl.ds(s * ZROWS, ZROWS)])
            plsc.subcore_barrier()

            def chunk(j, carry):
                pltpu.sync_copy(src_idx.at[p, c, s, j], sidx)
                pltpu.sync_copy(dst_idx.at[s, j], didx)
                pltpu.async_copy(table.at[sidx], rows, sem).wait()
                pltpu.sync_copy(rows, acc.at[didx], add=True)
                return carry

            lax.fori_loop(0, NJ16, chunk, 0)
            plsc.subcore_barrier()
            pltpu.sync_copy(acc.at[pl.ds(s * OROWS, OROWS)],
                            out.at[p * 2 + c, pl.ds(s * OROWS, OROWS)])
            if p + 1 < n_passes:
                plsc.subcore_barrier()

    return body


def _segsum_slabs(src_idx, dst_idx, table, n_passes, d):
    zeros = jnp.zeros((ZROWS, d), jnp.float32)
    mesh = plsc.VectorSubcoreMesh(core_axis_name="c", subcore_axis_name="s")
    return pl.kernel(
        _segsum_slab_body(n_passes, d),
        out_type=jax.ShapeDtypeStruct((n_passes * 2, OUT_ROWS, d), jnp.float32),
        mesh=mesh,
        compiler_params=pltpu.CompilerParams(use_tc_tiling_on_sc=False),
        scratch_types=[
            pltpu.VMEM_SHARED((ACC_ROWS, d), jnp.float32),
            pltpu.VMEM((CHUNK,), jnp.int32),
            pltpu.VMEM((CHUNK,), jnp.int32),
            pltpu.VMEM((CHUNK, d), jnp.float32),
            pltpu.SemaphoreType.DMA,
        ],
    )(src_idx, dst_idx, table, zeros)


def _segsum8_body(src_idx, dst_idx, table, zeros, out, acc, sidx, didx, rows, sem):
    c = lax.axis_index("c")
    s = lax.axis_index("s")
    w = c * 16 + s
    pltpu.sync_copy(zeros, acc.at[pl.ds(s * ZROWS, ZROWS)])
    plsc.subcore_barrier()

    def chunk(j, carry):
        pltpu.sync_copy(src_idx.at[w, j], sidx)
        pltpu.sync_copy(dst_idx.at[w, j], didx)
        pltpu.async_copy(table.at[sidx], rows, sem).wait()
        pltpu.sync_copy(rows, acc.at[didx], add=True)
        return carry

    lax.fori_loop(0, NJ32, chunk, 0)
    plsc.subcore_barrier()
    pltpu.sync_copy(acc.at[pl.ds(s * OROWS, OROWS)],
                    out.at[c, pl.ds(s * OROWS, OROWS)])


def _segsum8(src_idx, dst_idx, table):
    zeros = jnp.zeros((ZROWS, 8), jnp.float32)
    mesh = plsc.VectorSubcoreMesh(core_axis_name="c", subcore_axis_name="s")
    out = pl.kernel(
        _segsum8_body,
        out_type=jax.ShapeDtypeStruct((2, OUT_ROWS, 8), jnp.float32),
        mesh=mesh,
        compiler_params=pltpu.CompilerParams(use_tc_tiling_on_sc=False),
        scratch_types=[
            pltpu.VMEM_SHARED((ACC_ROWS, 8), jnp.float32),
            pltpu.VMEM((CHUNK,), jnp.int32),
            pltpu.VMEM((CHUNK,), jnp.int32),
            pltpu.VMEM((CHUNK, 8), jnp.float32),
            pltpu.SemaphoreType.DMA,
        ],
    )(src_idx, dst_idx, table, zeros)
    return out[0, :NSEG] + out[1, :NSEG]


def _build_indices(lit_idx, cls_idx):
    pad_n = E_PAD - N_EDGES
    ar = jnp.arange(pad_n, dtype=jnp.int32)
    lit_idx = lit_idx.astype(jnp.int32)
    cls_idx = cls_idx.astype(jnp.int32)
    pad_src = ar % 128
    pad_dst = NSEG + ar % (ACC_ROWS - NSEG)
    lit_s = jnp.concatenate([lit_idx, pad_src])
    lit_d = jnp.concatenate([lit_idx, pad_dst])
    cls_s = jnp.concatenate([cls_idx, pad_src])
    cls_d = jnp.concatenate([cls_idx, pad_dst])
    l16s = lit_s.reshape(16, NJ16, CHUNK)
    l16d = lit_d.reshape(16, NJ16, CHUNK)
    c16s = cls_s.reshape(16, NJ16, CHUNK)
    c16d = cls_d.reshape(16, NJ16, CHUNK)
    idx = {}
    idx["A_src"] = jnp.stack([l16s, l16s + NSEG])[None]
    idx["A_dst"] = c16d
    idx["B_src"] = jnp.stack(
        [jnp.stack([c16s, c16s + NSEG]),
         jnp.stack([c16s + 2 * NSEG, c16s + 3 * NSEG])])
    idx["B_dst"] = l16d
    idx["l32s"] = lit_s.reshape(32, NJ32, CHUNK)
    idx["l32d"] = lit_d.reshape(32, NJ32, CHUNK)
    idx["c32s"] = cls_s.reshape(32, NJ32, CHUNK)
    idx["c32d"] = cls_d.reshape(32, NJ32, CHUNK)
    return idx


def _pair_norm(x):
    x = x - jnp.mean(x, axis=0, keepdims=True)
    var = jnp.mean(jnp.sum(jnp.square(x), axis=-1))
    return x * jax.lax.rsqrt(var + EPS)


def _mlp2(x, W1, b1, W2, b2):
    return jax.nn.relu(x @ W1 + b1) @ W2 + b2


def _mlp3(x, W1, b1, W2, b2, W3, b3):
    h = jax.nn.relu(x @ W1 + b1)
    h = jax.nn.relu(h @ W2 + b2)
    return h @ W3 + b3


def _mixed_loss_kernel(s8_ref, out_ref):
    cv = jnp.exp(-s8_ref[...])
    mixed = cv * (-jnp.log(1.0 - cv + EPS))
    out_ref[...] = jnp.sum(mixed, axis=1, keepdims=True) / float(TEST_ROUNDS)


def _mixed_loss(S8):
    blk = 5000
    return pl.pallas_call(
        _mixed_loss_kernel,
        grid=(N_CLAUSES // blk,),
        in_specs=[pl.BlockSpec((blk, 8), lambda i: (i, 0))],
        out_specs=pl.BlockSpec((blk, 1), lambda i: (i, 0)),
        out_shape=jax.ShapeDtypeStruct((N_CLAUSES, 1), jnp.float32),
    )(S8)


def kernel(edge_index, Wq1, bq1, Wq2, bq2, Wc1, bc1, Wc2, bc2, Wu1, bu1, Wu2, bu2, Wu3, bu3, Wo1, bo1, Wo2, bo2):
    idx = _build_indices(edge_index[0], edge_index[1])

    ones8 = jnp.ones((NSEG, 8), jnp.float32)
    lit_degree = _segsum8(idx["c32s"], idx["l32d"], ones8)[:, 0:1]
    degree_weight = jax.lax.rsqrt(jnp.maximum(lit_degree, 1.0))
    var_degree_weight = 4.0 * jax.lax.rsqrt(
        jnp.maximum(lit_degree[:N_VARS] + lit_degree[N_VARS:], 1.0))

    variables = jnp.ones((N_VARS, FEATURE_MAPS), jnp.float32)
    clause_state = jnp.ones((N_CLAUSES, FEATURE_MAPS), jnp.float32)
    nkey = jax.random.key(42)
    last_logits = jnp.zeros((N_VARS, 1), jnp.float32)
    l8_cols = []
    for step in range(TEST_ROUNDS):
        noise = jax.random.normal(jax.random.fold_in(nkey, step), (N_VARS, 4), dtype=jnp.float32)
        v1 = jnp.concatenate([variables, noise], axis=-1)
        query = _mlp2(v1, Wq1, bq1, Wq2, bq2)
        lits = jax.nn.softplus(jnp.concatenate([query, -query], axis=0))
        tabA = jnp.concatenate([lits[:, :32], lits[:, 32:]], axis=0)
        outA = _segsum_slabs(idx["A_src"], idx["A_dst"], tabA, 1, 32)
        S = jnp.concatenate([outA[0, :NSEG], outA[1, :NSEG]], axis=1)
        cl = jnp.exp(-S)
        clauses_loss = cl * 4.0
        clause_unit = jnp.concatenate([clause_state, clauses_loss], axis=-1)
        clause_data = _mlp2(clause_unit, Wc1, bc1, Wc2, bc2)
        variables_loss_all = clause_data[:, :QUERY_MAPS]
        new_clause_value = _pair_norm(clause_data[:, QUERY_MAPS:]) * 0.25
        clause_state = new_clause_value + 0.1 * clause_state
        tabB = jnp.concatenate(
            [cl[:, :32], cl[:, 32:],
             variables_loss_all[:, :32], variables_loss_all[:, 32:]], axis=0)
        outB = _segsum_slabs(idx["B_src"], idx["B_dst"], tabB, 2, 32)
        g = -jnp.concatenate([outB[0, :NSEG], outB[1, :NSEG]], axis=1)
        variables_loss = jnp.concatenate([outB[2, :NSEG], outB[3, :NSEG]], axis=1) * degree_weight
        variables_grad = (g[:N_VARS] * jax.nn.sigmoid(query)
                          - g[N_VARS:] * jax.nn.sigmoid(-query)) * var_degree_weight
        unit = jnp.concatenate(
            [variables_grad, variables, variables_loss[:N_VARS], variables_loss[N_VARS:]], axis=-1)
        new_variables = _pair_norm(_mlp3(unit, Wu1, bu1, Wu2, bu2, Wu3, bu3)) * 0.25
        variables = new_variables + 0.1 * variables
        logits = _mlp2(variables, Wo1, bo1, Wo2, bo2)
        l8_cols.append(jax.nn.softplus(jnp.concatenate([logits, -logits], axis=0))[:, 0])
        last_logits = logits
    L8 = jnp.stack(l8_cols, axis=1)
    S8 = _segsum8(idx["l32s"], idx["c32d"], L8)
    unsupervised_loss = _mixed_loss(S8)
    step = jnp.asarray(TEST_ROUNDS - 1, dtype=jnp.int32)
    return (last_logits, unsupervised_loss, step)


# speedup vs baseline: 5.1583x; 1.0494x over previous
import functools

import jax
import jax.numpy as jnp
from jax import lax
from jax.experimental import pallas as pl
from jax.experimental.pallas import tpu as pltpu
from jax.experimental.pallas import tpu_sc as plsc

N_VARS = 25000
N_CLAUSES = 50000
N_EDGES = 800000
FEATURE_MAPS = 64
QUERY_MAPS = 64
TEST_ROUNDS = 8
EPS = 1e-6

NSEG = 50000
E_PAD = 802816
CHUNK = 128
NJ16 = E_PAD // (16 * CHUNK)
NJ32 = E_PAD // (32 * CHUNK)
ACC_ROWS = 50176
ZROWS = ACC_ROWS // 16
OUT_ROWS = 50048
OROWS = OUT_ROWS // 16


WIN = 4


def _ring_chunks(nj, sref, dref, table, acc, sidx, didx, rows, sem):
    def window(w, carry):
        pltpu.sync_copy(sref(w), sidx)
        for k in range(WIN):
            pltpu.sync_copy(dref(w, k), didx[k])
        for k in range(WIN):
            pltpu.sync_copy(table.at[sidx.at[k]], rows[k])
        for k in range(WIN):
            pltpu.sync_copy(rows[k], acc.at[didx[k]], add=True)
        return carry

    lax.fori_loop(0, nj // WIN, window, 0)


def _segsum_slab_body(n_passes, d):
    def body(src_idx, dst_idx, table, zeros, out, acc,
             sidx, d0, d1, d2, d3, r0, r1, r2, r3, sem):
        c = lax.axis_index("c")
        s = lax.axis_index("s")
        for p in range(n_passes):
            pltpu.sync_copy(zeros, acc.at[pl.ds(s * ZROWS, ZROWS)])
            plsc.subcore_barrier()
            _ring_chunks(NJ16,
                         lambda w: src_idx.at[p, c, s, pl.ds(w * WIN, WIN)],
                         lambda w, k: dst_idx.at[s, w * WIN + k],
                         table, acc, sidx, [d0, d1, d2, d3],
                         [r0, r1, r2, r3], sem)
            plsc.subcore_barrier()
            pltpu.sync_copy(acc.at[pl.ds(s * OROWS, OROWS)],
                            out.at[p * 2 + c, pl.ds(s * OROWS, OROWS)])
            if p + 1 < n_passes:
                plsc.subcore_barrier()

    return body


def _segsum_slabs(src_idx, dst_idx, table, n_passes, d):
    zeros = jnp.zeros((ZROWS, d), jnp.float32)
    mesh = plsc.VectorSubcoreMesh(core_axis_name="c", subcore_axis_name="s")
    return pl.kernel(
        _segsum_slab_body(n_passes, d),
        out_type=jax.ShapeDtypeStruct((n_passes * 2, OUT_ROWS, d), jnp.float32),
        mesh=mesh,
        compiler_params=pltpu.CompilerParams(use_tc_tiling_on_sc=False),
        scratch_types=_ring_scratch(d),
    )(src_idx, dst_idx, table, zeros)


def _ring_scratch(d):
    return ([
        pltpu.VMEM_SHARED((ACC_ROWS, d), jnp.float32),
        pltpu.VMEM((WIN, CHUNK), jnp.int32),
    ] + [pltpu.VMEM((CHUNK,), jnp.int32) for _ in range(WIN)]
      + [pltpu.VMEM((CHUNK, d), jnp.float32) for _ in range(WIN)]
      + [pltpu.SemaphoreType.DMA])


def _segsum8_body(src_idx, dst_idx, table, zeros, out, acc,
                  sidx, d0, d1, d2, d3, r0, r1, r2, r3, sem):
    c = lax.axis_index("c")
    s = lax.axis_index("s")
    wid = c * 16 + s
    pltpu.sync_copy(zeros, acc.at[pl.ds(s * ZROWS, ZROWS)])
    plsc.subcore_barrier()
    _ring_chunks(NJ32,
                 lambda w: src_idx.at[wid, pl.ds(w * WIN, WIN)],
                 lambda w, k: dst_idx.at[wid, w * WIN + k],
                 table, acc, sidx, [d0, d1, d2, d3],
                 [r0, r1, r2, r3], sem)
    plsc.subcore_barrier()
    pltpu.sync_copy(acc.at[pl.ds(s * OROWS, OROWS)],
                    out.at[c, pl.ds(s * OROWS, OROWS)])


def _segsum8(src_idx, dst_idx, table):
    zeros = jnp.zeros((ZROWS, 8), jnp.float32)
    mesh = plsc.VectorSubcoreMesh(core_axis_name="c", subcore_axis_name="s")
    out = pl.kernel(
        _segsum8_body,
        out_type=jax.ShapeDtypeStruct((2, OUT_ROWS, 8), jnp.float32),
        mesh=mesh,
        compiler_params=pltpu.CompilerParams(use_tc_tiling_on_sc=False),
        scratch_types=_ring_scratch(8),
    )(src_idx, dst_idx, table, zeros)
    return out[0, :NSEG] + out[1, :NSEG]


def _build_indices(lit_idx, cls_idx):
    pad_n = E_PAD - N_EDGES
    ar = jnp.arange(pad_n, dtype=jnp.int32)
    lit_idx = lit_idx.astype(jnp.int32)
    cls_idx = cls_idx.astype(jnp.int32)
    pad_src = ar % 128
    pad_dst = NSEG + ar % (ACC_ROWS - NSEG)
    perm_c = jnp.argsort(cls_idx, stable=True)
    perm_l = jnp.argsort(lit_idx, stable=True)
    a_s = jnp.concatenate([lit_idx[perm_c], pad_src])
    a_d = jnp.concatenate([cls_idx[perm_c], pad_dst])
    b_s = jnp.concatenate([cls_idx[perm_l], pad_src])
    b_d = jnp.concatenate([lit_idx[perm_l], pad_dst])
    def cm(x):
        return x.reshape(16, CHUNK, NJ16).transpose(0, 2, 1)
    a_s16 = cm(a_s)
    a_d16 = cm(a_d)
    b_s16 = cm(b_s)
    b_d16 = cm(b_d)
    idx = {}
    idx["A_src"] = jnp.stack([a_s16, a_s16 + NSEG])[None]
    idx["A_dst"] = a_d16
    idx["B_src"] = jnp.stack(
        [jnp.stack([b_s16, b_s16 + NSEG]),
         jnp.stack([b_s16 + 2 * NSEG, b_s16 + 3 * NSEG])])
    idx["B_dst"] = b_d16
    lit_s = jnp.concatenate([lit_idx, pad_src])
    lit_d = jnp.concatenate([lit_idx, pad_dst])
    cls_s = jnp.concatenate([cls_idx, pad_src])
    cls_d = jnp.concatenate([cls_idx, pad_dst])
    idx["l32s"] = lit_s.reshape(32, NJ32, CHUNK)
    idx["l32d"] = lit_d.reshape(32, NJ32, CHUNK)
    idx["c32s"] = cls_s.reshape(32, NJ32, CHUNK)
    idx["c32d"] = cls_d.reshape(32, NJ32, CHUNK)
    return idx


def _pair_norm(x):
    x = x - jnp.mean(x, axis=0, keepdims=True)
    var = jnp.mean(jnp.sum(jnp.square(x), axis=-1))
    return x * jax.lax.rsqrt(var + EPS)


def _mlp2(x, W1, b1, W2, b2):
    return jax.nn.relu(x @ W1 + b1) @ W2 + b2


def _mlp3(x, W1, b1, W2, b2, W3, b3):
    h = jax.nn.relu(x @ W1 + b1)
    h = jax.nn.relu(h @ W2 + b2)
    return h @ W3 + b3


def _mixed_loss_kernel(s8_ref, out_ref):
    cv = jnp.exp(-s8_ref[...])
    mixed = cv * (-jnp.log(1.0 - cv + EPS))
    out_ref[...] = jnp.sum(mixed, axis=1, keepdims=True) / float(TEST_ROUNDS)


def _mixed_loss(S8):
    blk = 5000
    return pl.pallas_call(
        _mixed_loss_kernel,
        grid=(N_CLAUSES // blk,),
        in_specs=[pl.BlockSpec((blk, 8), lambda i: (i, 0))],
        out_specs=pl.BlockSpec((blk, 1), lambda i: (i, 0)),
        out_shape=jax.ShapeDtypeStruct((N_CLAUSES, 1), jnp.float32),
    )(S8)


def kernel(edge_index, Wq1, bq1, Wq2, bq2, Wc1, bc1, Wc2, bc2, Wu1, bu1, Wu2, bu2, Wu3, bu3, Wo1, bo1, Wo2, bo2):
    idx = _build_indices(edge_index[0], edge_index[1])

    ones8 = jnp.ones((NSEG, 8), jnp.float32)
    lit_degree = _segsum8(idx["c32s"], idx["l32d"], ones8)[:, 0:1]
    degree_weight = jax.lax.rsqrt(jnp.maximum(lit_degree, 1.0))
    var_degree_weight = 4.0 * jax.lax.rsqrt(
        jnp.maximum(lit_degree[:N_VARS] + lit_degree[N_VARS:], 1.0))

    variables = jnp.ones((N_VARS, FEATURE_MAPS), jnp.float32)
    clause_state = jnp.ones((N_CLAUSES, FEATURE_MAPS), jnp.float32)
    nkey = jax.random.key(42)
    last_logits = jnp.zeros((N_VARS, 1), jnp.float32)
    l8_cols = []
    for step in range(TEST_ROUNDS):
        noise = jax.random.normal(jax.random.fold_in(nkey, step), (N_VARS, 4), dtype=jnp.float32)
        v1 = jnp.concatenate([variables, noise], axis=-1)
        query = _mlp2(v1, Wq1, bq1, Wq2, bq2)
        lits = jax.nn.softplus(jnp.concatenate([query, -query], axis=0))
        tabA = jnp.concatenate([lits[:, :32], lits[:, 32:]], axis=0)
        outA = _segsum_slabs(idx["A_src"], idx["A_dst"], tabA, 1, 32)
        S = jnp.concatenate([outA[0, :NSEG], outA[1, :NSEG]], axis=1)
        cl = jnp.exp(-S)
        clauses_loss = cl * 4.0
        clause_unit = jnp.concatenate([clause_state, clauses_loss], axis=-1)
        clause_data = _mlp2(clause_unit, Wc1, bc1, Wc2, bc2)
        variables_loss_all = clause_data[:, :QUERY_MAPS]
        new_clause_value = _pair_norm(clause_data[:, QUERY_MAPS:]) * 0.25
        clause_state = new_clause_value + 0.1 * clause_state
        tabB = jnp.concatenate(
            [cl[:, :32], cl[:, 32:],
             variables_loss_all[:, :32], variables_loss_all[:, 32:]], axis=0)
        outB = _segsum_slabs(idx["B_src"], idx["B_dst"], tabB, 2, 32)
        g = -jnp.concatenate([outB[0, :NSEG], outB[1, :NSEG]], axis=1)
        variables_loss = jnp.concatenate([outB[2, :NSEG], outB[3, :NSEG]], axis=1) * degree_weight
        variables_grad = (g[:N_VARS] * jnp.exp(query - lits[:N_VARS])
                          - g[N_VARS:] * jnp.exp(-query - lits[N_VARS:])) * var_degree_weight
        unit = jnp.concatenate(
            [variables_grad, variables, variables_loss[:N_VARS], variables_loss[N_VARS:]], axis=-1)
        new_variables = _pair_norm(_mlp3(unit, Wu1, bu1, Wu2, bu2, Wu3, bu3)) * 0.25
        variables = new_variables + 0.1 * variables
        logits = _mlp2(variables, Wo1, bo1, Wo2, bo2)
        l8_cols.append(jax.nn.softplus(jnp.concatenate([logits, -logits], axis=0))[:, 0])
        last_logits = logits
        variables = variables * 0.2 + variables * 0.8
        clause_state = clause_state * 0.2 + clause_state * 0.8
    L8 = jnp.stack(l8_cols, axis=1)
    S8 = _segsum8(idx["l32s"], idx["c32d"], L8)
    unsupervised_loss = _mixed_loss(S8)
    step = jnp.asarray(TEST_ROUNDS - 1, dtype=jnp.int32)
    return (last_logits, unsupervised_loss, step)
